# Initial kernel scaffold; baseline (speedup 1.0000x reference)
import jax
import jax.numpy as jnp
from jax.experimental import pallas as pl


def kernel(data, edge_index, edge_attr, g_W0, mu0, sigma0, root_W0, bias0, g_W1, mu1, sigma1, root_W1, bias1, fc_W, fc_b):
    raise NotImplementedError("write your pallas kernel here")


# speedup vs baseline: 1.7458x; 1.7458x over previous
import functools

import jax
import jax.numpy as jnp
from jax import lax
from jax.experimental import pallas as pl
from jax.experimental.pallas import tpu as pltpu
from jax.experimental.pallas import tpu_sc as plsc

NC = 2
NS = 16
LANES = 16
NW = NC * NS

K = 3
H = 64
AW = 128
XGW = 256


def _edge_weights_body(attr_ref, mu0_ref, sig0_ref, mu1_ref, sig1_ref,
                       w0_ref, w1_ref):
    a0 = attr_ref[:, 0:1]
    a1 = attr_ref[:, 1:2]
    eb = a0.shape[0]
    for mu_ref, sig_ref, out_ref in ((mu0_ref, sig0_ref, w0_ref),
                                     (mu1_ref, sig1_ref, w1_ref)):
        for k in range(K):
            d0 = a0 - mu_ref[k, 0]
            d1 = a1 - mu_ref[k, 1]
            q = -0.5 * (d0 * d0 / (1e-15 + sig_ref[k, 0] ** 2)
                        + d1 * d1 / (1e-15 + sig_ref[k, 1] ** 2))
            wk = jnp.exp(q)
            out_ref[:, k * LANES:(k + 1) * LANES] = jnp.broadcast_to(
                wk, (eb, LANES))


def _edge_weights(edge_attr, mu0, sigma0, mu1, sigma1):
    e = edge_attr.shape[0]
    eb = 8000
    grid = e // eb
    smem = pl.BlockSpec(memory_space=pltpu.MemorySpace.SMEM)
    return pl.pallas_call(
        _edge_weights_body,
        grid=(grid,),
        in_specs=[
            pl.BlockSpec((eb, 2), lambda i: (i, 0)),
            smem, smem, smem, smem,
        ],
        out_specs=[
            pl.BlockSpec((eb, K * LANES), lambda i: (i, 0)),
            pl.BlockSpec((eb, K * LANES), lambda i: (i, 0)),
        ],
        out_shape=[
            jax.ShapeDtypeStruct((e, K * LANES), jnp.float32),
            jax.ShapeDtypeStruct((e, K * LANES), jnp.float32),
        ],
    )(edge_attr, mu0, sigma0, mu1, sigma1)


def _node_pre_body(x_ref, gw_ref, rw_ref, xg_ref, root_ref):
    x = x_ref[...]
    n = x.shape[0]
    xg_ref[:, 0:K * H] = jnp.dot(x, gw_ref[...],
                                 preferred_element_type=jnp.float32)
    xg_ref[:, K * H:] = jnp.zeros((n, XGW - K * H), jnp.float32)
    root_ref[...] = jnp.dot(x, rw_ref[...], preferred_element_type=jnp.float32)


def _node_pre(x, g_w, root_w):
    n = x.shape[0]
    return pl.pallas_call(
        _node_pre_body,
        out_shape=[
            jax.ShapeDtypeStruct((n, XGW), jnp.float32),
            jax.ShapeDtypeStruct((n, H), jnp.float32),
        ],
    )(x, g_w, root_w)


def _combine_pre_body(part_ref, root_ref, bias_ref, gw_ref, rw_ref,
                      xg_ref, root1_ref):
    n = root_ref.shape[0]
    s = part_ref[0, :n] + part_ref[1, :n]
    cnt = jnp.maximum(s[:, H:H + 1], 1.0)
    aggr = s[:, 0:H] / cnt
    x1 = jnp.maximum(aggr + root_ref[...] + bias_ref[...], 0.0)
    xg_ref[:, 0:K * H] = jnp.dot(x1, gw_ref[...],
                                 preferred_element_type=jnp.float32)
    xg_ref[:, K * H:] = jnp.zeros((n, XGW - K * H), jnp.float32)
    root1_ref[...] = jnp.dot(x1, rw_ref[...],
                             preferred_element_type=jnp.float32)


def _combine_pre(part, root, bias, g_w, root_w):
    n = root.shape[0]
    return pl.pallas_call(
        _combine_pre_body,
        out_shape=[
            jax.ShapeDtypeStruct((n, XGW), jnp.float32),
            jax.ShapeDtypeStruct((n, H), jnp.float32),
        ],
    )(part, root, bias.reshape(1, H), g_w, root_w)


def _final_body(part_ref, root_ref, bias_ref, fcw_ref, fcb_ref, out_ref):
    n = root_ref.shape[0]
    s = part_ref[0, :n] + part_ref[1, :n]
    cnt = jnp.maximum(s[:, H:H + 1], 1.0)
    aggr = s[:, 0:H] / cnt
    x2 = jnp.maximum(aggr + root_ref[...] + bias_ref[...], 0.0)
    logits = (jnp.dot(x2, fcw_ref[...], preferred_element_type=jnp.float32)
              + fcb_ref[...])
    m = jnp.max(logits, axis=1, keepdims=True)
    z = logits - m
    lse = jnp.log(jnp.sum(jnp.exp(z), axis=1, keepdims=True))
    out_ref[...] = z - lse


def _final(part, root, bias, fc_w, fc_b):
    n = root.shape[0]
    c = fc_w.shape[1]
    return pl.pallas_call(
        _final_body,
        out_shape=jax.ShapeDtypeStruct((n, c), jnp.float32),
    )(part, root, bias.reshape(1, H), fc_w, fc_b.reshape(1, c))


EDGE_BATCH = 80


def _sc_aggregate_body(rows_per_tile, n_batches, edges_per_tile,
                       xg_hbm, src_hbm, dst_hbm, w_hbm, out_hbm,
                       src_v, dst_v, w_v, g_v, msg_v, acc_sh, sem):
    cid = lax.axis_index("c")
    sid = lax.axis_index("s")
    wid = cid * NS + sid
    b = EDGE_BATCH

    zeros = jnp.zeros((LANES,), jnp.float32)
    ones = jnp.ones((LANES,), jnp.float32)

    def zero_row(r, carry):
        for c in range(AW // LANES):
            msg_v[r, pl.ds(c * LANES, LANES)] = zeros
        return carry

    lax.fori_loop(0, b, zero_row, 0)

    row0 = sid * rows_per_tile

    def zero_chunk(c2, carry):
        pltpu.sync_copy(msg_v.at[pl.ds(0, 8)],
                        acc_sh.at[pl.ds(row0 + c2 * 8, 8)])
        return carry

    lax.fori_loop(0, rows_per_tile // 8, zero_chunk, 0)

    def init_cnt_row(r, carry):
        msg_v[r, pl.ds(H, LANES)] = ones
        return carry

    lax.fori_loop(0, b, init_cnt_row, 0)
    plsc.subcore_barrier()

    base0 = wid * edges_per_tile

    def batch(i, carry):
        base = base0 + i * b
        pltpu.sync_copy(src_hbm.at[pl.ds(base, b)], src_v)
        pltpu.sync_copy(dst_hbm.at[pl.ds(base, b)], dst_v)
        pltpu.sync_copy(w_hbm.at[pl.ds(base, b)], w_v)
        pltpu.async_copy(xg_hbm.at[src_v], g_v, sem).wait()

        def edge(j, c2):
            w0 = w_v[j, pl.ds(0, LANES)]
            w1 = w_v[j, pl.ds(LANES, LANES)]
            w2 = w_v[j, pl.ds(2 * LANES, LANES)]
            for c in range(H // LANES):
                g0 = g_v[j, pl.ds(c * LANES, LANES)]
                g1 = g_v[j, pl.ds(H + c * LANES, LANES)]
                g2 = g_v[j, pl.ds(2 * H + c * LANES, LANES)]
                msg_v[j, pl.ds(c * LANES, LANES)] = g0 * w0 + g1 * w1 + g2 * w2
            return c2

        lax.fori_loop(0, b, edge, 0)
        pltpu.sync_copy(msg_v, acc_sh.at[dst_v], add=True)
        return carry

    lax.fori_loop(0, n_batches, batch, 0)
    plsc.subcore_barrier()

    def dump_chunk(c2, carry):
        pltpu.sync_copy(acc_sh.at[pl.ds(row0 + c2 * 8, 8)],
                        msg_v.at[pl.ds(0, 8)])
        pltpu.sync_copy(msg_v.at[pl.ds(0, 8)],
                        out_hbm.at[cid, pl.ds(row0 + c2 * 8, 8)])
        return carry

    lax.fori_loop(0, rows_per_tile // 8, dump_chunk, 0)


def _sc_aggregate(xg, src, dst, w):
    n = xg.shape[0]
    e = src.shape[0]
    edges_per_tile = e // NW
    n_batches = edges_per_tile // EDGE_BATCH
    n_pad = -(-n // (NS * 8)) * (NS * 8)
    rows_per_tile = n_pad // NS
    mesh = plsc.VectorSubcoreMesh(core_axis_name="c", subcore_axis_name="s",
                                  num_cores=NC, num_subcores=NS)
    body = functools.partial(_sc_aggregate_body, rows_per_tile, n_batches,
                             edges_per_tile)
    return pl.kernel(
        body,
        out_type=jax.ShapeDtypeStruct((NC, n_pad, AW), jnp.float32),
        mesh=mesh,
        scratch_types=[
            pltpu.VMEM((EDGE_BATCH,), jnp.int32),
            pltpu.VMEM((EDGE_BATCH,), jnp.int32),
            pltpu.VMEM((EDGE_BATCH, K * LANES), jnp.float32),
            pltpu.VMEM((EDGE_BATCH, XGW), jnp.float32),
            pltpu.VMEM((EDGE_BATCH, AW), jnp.float32),
            pltpu.VMEM_SHARED((n_pad, AW), jnp.float32),
            pltpu.SemaphoreType.DMA,
        ],
    )(xg, src, dst, w)


def kernel(data, edge_index, edge_attr, g_W0, mu0, sigma0, root_W0, bias0,
           g_W1, mu1, sigma1, root_W1, bias1, fc_W, fc_b):
    src = edge_index[0]
    dst = edge_index[1]
    w0x, w1x = _edge_weights(edge_attr, mu0, sigma0, mu1, sigma1)
    xg0, root0 = _node_pre(data, g_W0, root_W0)
    part0 = _sc_aggregate(xg0, src, dst, w0x)
    xg1, root1 = _combine_pre(part0, root0, bias0, g_W1, root_W1)
    part1 = _sc_aggregate(xg1, src, dst, w1x)
    return _final(part1, root1, bias1, fc_W, fc_b)


# speedup vs baseline: 1.7775x; 1.0182x over previous
import functools

import jax
import jax.numpy as jnp
from jax import lax
from jax.experimental import pallas as pl
from jax.experimental.pallas import tpu as pltpu
from jax.experimental.pallas import tpu_sc as plsc

NC = 2
NS = 16
LANES = 16
NW = NC * NS

K = 3
H = 64
AW = 128
XGW = 256


def _edge_weights_body(attr_ref, mu0_ref, sig0_ref, mu1_ref, sig1_ref,
                       w0_ref, w1_ref):
    a0 = attr_ref[:, 0:1]
    a1 = attr_ref[:, 1:2]
    eb = a0.shape[0]
    for mu_ref, sig_ref, out_ref in ((mu0_ref, sig0_ref, w0_ref),
                                     (mu1_ref, sig1_ref, w1_ref)):
        for k in range(K):
            d0 = a0 - mu_ref[k, 0]
            d1 = a1 - mu_ref[k, 1]
            q = -0.5 * (d0 * d0 / (1e-15 + sig_ref[k, 0] ** 2)
                        + d1 * d1 / (1e-15 + sig_ref[k, 1] ** 2))
            out_ref[:, k:k + 1] = jnp.exp(q)
        out_ref[:, K:K + 1] = jnp.ones((eb, 1), jnp.float32)


def _edge_weights(edge_attr, mu0, sigma0, mu1, sigma1):
    e = edge_attr.shape[0]
    eb = 8000
    grid = e // eb
    smem = pl.BlockSpec(memory_space=pltpu.MemorySpace.SMEM)
    return pl.pallas_call(
        _edge_weights_body,
        grid=(grid,),
        in_specs=[
            pl.BlockSpec((eb, 2), lambda i: (i, 0)),
            smem, smem, smem, smem,
        ],
        out_specs=[
            pl.BlockSpec((eb, K + 1), lambda i: (i, 0)),
            pl.BlockSpec((eb, K + 1), lambda i: (i, 0)),
        ],
        out_shape=[
            jax.ShapeDtypeStruct((e, K + 1), jnp.float32),
            jax.ShapeDtypeStruct((e, K + 1), jnp.float32),
        ],
    )(edge_attr, mu0, sigma0, mu1, sigma1)


def _node_pre_body(x_ref, gw_ref, rw_ref, xg_ref, root_ref):
    x = x_ref[...]
    n = x.shape[0]
    xg_ref[:, 0:K * H] = jnp.dot(x, gw_ref[...],
                                 preferred_element_type=jnp.float32)
    xg_ref[:, K * H:] = jnp.zeros((n, XGW - K * H), jnp.float32)
    root_ref[...] = jnp.dot(x, rw_ref[...], preferred_element_type=jnp.float32)


def _node_pre(x, g_w, root_w):
    n = x.shape[0]
    return pl.pallas_call(
        _node_pre_body,
        out_shape=[
            jax.ShapeDtypeStruct((n, XGW), jnp.float32),
            jax.ShapeDtypeStruct((n, H), jnp.float32),
        ],
    )(x, g_w, root_w)


def _combine_pre_body(part_ref, root_ref, bias_ref, gw_ref, rw_ref,
                      xg_ref, root1_ref):
    n = root_ref.shape[0]
    s = part_ref[0, :n] + part_ref[1, :n]
    cnt = jnp.maximum(s[:, H:H + 1], 1.0)
    aggr = s[:, 0:H] / cnt
    x1 = jnp.maximum(aggr + root_ref[...] + bias_ref[...], 0.0)
    xg_ref[:, 0:K * H] = jnp.dot(x1, gw_ref[...],
                                 preferred_element_type=jnp.float32)
    xg_ref[:, K * H:] = jnp.zeros((n, XGW - K * H), jnp.float32)
    root1_ref[...] = jnp.dot(x1, rw_ref[...],
                             preferred_element_type=jnp.float32)


def _combine_pre(part, root, bias, g_w, root_w):
    n = root.shape[0]
    return pl.pallas_call(
        _combine_pre_body,
        out_shape=[
            jax.ShapeDtypeStruct((n, XGW), jnp.float32),
            jax.ShapeDtypeStruct((n, H), jnp.float32),
        ],
    )(part, root, bias.reshape(1, H), g_w, root_w)


def _final_body(part_ref, root_ref, bias_ref, fcw_ref, fcb_ref, out_ref):
    n = root_ref.shape[0]
    s = part_ref[0, :n] + part_ref[1, :n]
    cnt = jnp.maximum(s[:, H:H + 1], 1.0)
    aggr = s[:, 0:H] / cnt
    x2 = jnp.maximum(aggr + root_ref[...] + bias_ref[...], 0.0)
    logits = (jnp.dot(x2, fcw_ref[...], preferred_element_type=jnp.float32)
              + fcb_ref[...])
    m = jnp.max(logits, axis=1, keepdims=True)
    z = logits - m
    lse = jnp.log(jnp.sum(jnp.exp(z), axis=1, keepdims=True))
    out_ref[...] = z - lse


def _final(part, root, bias, fc_w, fc_b):
    n = root.shape[0]
    c = fc_w.shape[1]
    return pl.pallas_call(
        _final_body,
        out_shape=jax.ShapeDtypeStruct((n, c), jnp.float32),
    )(part, root, bias.reshape(1, H), fc_w, fc_b.reshape(1, c))


EDGE_BATCH = 80
SB = 8
BPT = 128


def _sc_aggregate_body(rows_per_tile, n_valid_batches,
                       xg_hbm, src_hbm, dst_hbm, w_hbm, out_hbm,
                       src_sb, dst_v, w_sb, g_v, msg_v, acc_sh, sem):
    cid = lax.axis_index("c")
    sid = lax.axis_index("s")
    wid = cid * NS + sid
    b = EDGE_BATCH

    zeros = jnp.zeros((LANES,), jnp.float32)
    ones = jnp.ones((LANES,), jnp.float32)

    def zero_row(r, carry):
        for c in range(AW // LANES):
            msg_v[r, pl.ds(c * LANES, LANES)] = zeros
        return carry

    lax.fori_loop(0, b, zero_row, 0)

    row0 = sid * rows_per_tile

    def zero_chunk(c2, carry):
        pltpu.sync_copy(msg_v.at[pl.ds(0, 8)],
                        acc_sh.at[pl.ds(row0 + c2 * 8, 8)])
        return carry

    lax.fori_loop(0, rows_per_tile // 8, zero_chunk, 0)

    def init_cnt_row(r, carry):
        msg_v[r, pl.ds(H, LANES)] = ones
        return carry

    lax.fori_loop(0, b, init_cnt_row, 0)
    plsc.subcore_barrier()

    def super_batch(i, carry):
        r0 = wid * BPT + i * SB
        pltpu.sync_copy(src_hbm.at[pl.ds(r0, SB)], src_sb)
        pltpu.sync_copy(w_hbm.at[pl.ds(r0 * b * 4, SB * b * 4)], w_sb)
        for jj in range(SB):
            @pl.when(r0 + jj < n_valid_batches)
            def _batch():
                pltpu.sync_copy(dst_hbm.at[r0 + jj], dst_v)
                pltpu.async_copy(xg_hbm.at[src_sb.at[jj]], g_v, sem).wait()

                def group(q, c2):
                    wv = w_sb[pl.ds((jj * b + q * 4) * 4, LANES)]
                    for t in range(4):
                        j = q * 4 + t
                        w0 = jnp.broadcast_to(wv[4 * t + 0], (LANES,))
                        w1 = jnp.broadcast_to(wv[4 * t + 1], (LANES,))
                        w2 = jnp.broadcast_to(wv[4 * t + 2], (LANES,))
                        for c in range(H // LANES):
                            g0 = g_v[j, pl.ds(c * LANES, LANES)]
                            g1 = g_v[j, pl.ds(H + c * LANES, LANES)]
                            g2 = g_v[j, pl.ds(2 * H + c * LANES, LANES)]
                            msg_v[j, pl.ds(c * LANES, LANES)] = (
                                g0 * w0 + g1 * w1 + g2 * w2)
                    return c2

                lax.fori_loop(0, b // 4, group, 0)
                pltpu.sync_copy(msg_v, acc_sh.at[dst_v], add=True)
        return carry

    lax.fori_loop(0, BPT // SB, super_batch, 0)
    plsc.subcore_barrier()

    def dump_chunk(c2, carry):
        pltpu.sync_copy(acc_sh.at[pl.ds(row0 + c2 * 8, 8)],
                        msg_v.at[pl.ds(0, 8)])
        pltpu.sync_copy(msg_v.at[pl.ds(0, 8)],
                        out_hbm.at[cid, pl.ds(row0 + c2 * 8, 8)])
        return carry

    lax.fori_loop(0, rows_per_tile // 8, dump_chunk, 0)


def _sc_aggregate(xg, src, dst, w):
    n = xg.shape[0]
    e = src.shape[0]
    b = EDGE_BATCH
    n_valid_batches = e // b
    e_pad = NW * BPT * b
    src2d = jnp.pad(src, (0, e_pad - e)).reshape(e_pad // b, b)
    dst2d = jnp.pad(dst, (0, e_pad - e)).reshape(e_pad // b, b)
    w_pad = jnp.pad(w, ((0, e_pad - e), (0, 0))).reshape(-1)
    n_pad = -(-n // (NS * 8)) * (NS * 8)
    rows_per_tile = n_pad // NS
    mesh = plsc.VectorSubcoreMesh(core_axis_name="c", subcore_axis_name="s",
                                  num_cores=NC, num_subcores=NS)
    body = functools.partial(_sc_aggregate_body, rows_per_tile,
                             n_valid_batches)
    return pl.kernel(
        body,
        out_type=jax.ShapeDtypeStruct((NC, n_pad, AW), jnp.float32),
        mesh=mesh,
        scratch_types=[
            pltpu.VMEM((SB, b), jnp.int32),
            pltpu.VMEM((b,), jnp.int32),
            pltpu.VMEM((SB * b * (K + 1),), jnp.float32),
            pltpu.VMEM((b, XGW), jnp.float32),
            pltpu.VMEM((b, AW), jnp.float32),
            pltpu.VMEM_SHARED((n_pad, AW), jnp.float32),
            pltpu.SemaphoreType.DMA,
        ],
    )(xg, src2d, dst2d, w_pad)


def kernel(data, edge_index, edge_attr, g_W0, mu0, sigma0, root_W0, bias0,
           g_W1, mu1, sigma1, root_W1, bias1, fc_W, fc_b):
    src = edge_index[0]
    dst = edge_index[1]
    w0x, w1x = _edge_weights(edge_attr, mu0, sigma0, mu1, sigma1)
    xg0, root0 = _node_pre(data, g_W0, root_W0)
    part0 = _sc_aggregate(xg0, src, dst, w0x)
    xg1, root1 = _combine_pre(part0, root0, bias0, g_W1, root_W1)
    part1 = _sc_aggregate(xg1, src, dst, w1x)
    return _final(part1, root1, bias1, fc_W, fc_b)


# speedup vs baseline: 1.8898x; 1.0632x over previous
import functools

import jax
import jax.numpy as jnp
from jax import lax
from jax.experimental import pallas as pl
from jax.experimental.pallas import tpu as pltpu
from jax.experimental.pallas import tpu_sc as plsc

NC = 2
NS = 16
LANES = 16
NW = NC * NS

K = 3
H = 64
AW = 128
XGW = 256


def _edge_weights_body(attr_ref, mu0_ref, sig0_ref, mu1_ref, sig1_ref,
                       w0_ref, w1_ref):
    a0 = attr_ref[:, 0:1]
    a1 = attr_ref[:, 1:2]
    eb = a0.shape[0]
    for mu_ref, sig_ref, out_ref in ((mu0_ref, sig0_ref, w0_ref),
                                     (mu1_ref, sig1_ref, w1_ref)):
        for k in range(K):
            d0 = a0 - mu_ref[k, 0]
            d1 = a1 - mu_ref[k, 1]
            q = -0.5 * (d0 * d0 / (1e-15 + sig_ref[k, 0] ** 2)
                        + d1 * d1 / (1e-15 + sig_ref[k, 1] ** 2))
            out_ref[:, k:k + 1] = jnp.exp(q)
        out_ref[:, K:K + 1] = jnp.ones((eb, 1), jnp.float32)


def _edge_weights(edge_attr, mu0, sigma0, mu1, sigma1):
    e = edge_attr.shape[0]
    eb = 8000
    grid = e // eb
    smem = pl.BlockSpec(memory_space=pltpu.MemorySpace.SMEM)
    return pl.pallas_call(
        _edge_weights_body,
        grid=(grid,),
        in_specs=[
            pl.BlockSpec((eb, 2), lambda i: (i, 0)),
            smem, smem, smem, smem,
        ],
        out_specs=[
            pl.BlockSpec((eb, K + 1), lambda i: (i, 0)),
            pl.BlockSpec((eb, K + 1), lambda i: (i, 0)),
        ],
        out_shape=[
            jax.ShapeDtypeStruct((e, K + 1), jnp.float32),
            jax.ShapeDtypeStruct((e, K + 1), jnp.float32),
        ],
    )(edge_attr, mu0, sigma0, mu1, sigma1)


def _node_pre_body(x_ref, gw_ref, rw_ref, xg_ref, root_ref):
    x = x_ref[...]
    n = x.shape[0]
    xg_ref[:, 0:K * H] = jnp.dot(x, gw_ref[...],
                                 preferred_element_type=jnp.float32)
    xg_ref[:, K * H:] = jnp.zeros((n, XGW - K * H), jnp.float32)
    root_ref[...] = jnp.dot(x, rw_ref[...], preferred_element_type=jnp.float32)


def _node_pre(x, g_w, root_w):
    n = x.shape[0]
    return pl.pallas_call(
        _node_pre_body,
        out_shape=[
            jax.ShapeDtypeStruct((n, XGW), jnp.float32),
            jax.ShapeDtypeStruct((n, H), jnp.float32),
        ],
    )(x, g_w, root_w)


def _combine_pre_body(part_ref, root_ref, bias_ref, gw_ref, rw_ref,
                      xg_ref, root1_ref):
    n = root_ref.shape[0]
    s = part_ref[0, :n] + part_ref[1, :n]
    cnt = jnp.maximum(s[:, H:H + 1], 1.0)
    aggr = s[:, 0:H] / cnt
    x1 = jnp.maximum(aggr + root_ref[...] + bias_ref[...], 0.0)
    xg_ref[:, 0:K * H] = jnp.dot(x1, gw_ref[...],
                                 preferred_element_type=jnp.float32)
    xg_ref[:, K * H:] = jnp.zeros((n, XGW - K * H), jnp.float32)
    root1_ref[...] = jnp.dot(x1, rw_ref[...],
                             preferred_element_type=jnp.float32)


def _combine_pre(part, root, bias, g_w, root_w):
    n = root.shape[0]
    return pl.pallas_call(
        _combine_pre_body,
        out_shape=[
            jax.ShapeDtypeStruct((n, XGW), jnp.float32),
            jax.ShapeDtypeStruct((n, H), jnp.float32),
        ],
    )(part, root, bias.reshape(1, H), g_w, root_w)


def _final_body(part_ref, root_ref, bias_ref, fcw_ref, fcb_ref, out_ref):
    n = root_ref.shape[0]
    s = part_ref[0, :n] + part_ref[1, :n]
    cnt = jnp.maximum(s[:, H:H + 1], 1.0)
    aggr = s[:, 0:H] / cnt
    x2 = jnp.maximum(aggr + root_ref[...] + bias_ref[...], 0.0)
    logits = (jnp.dot(x2, fcw_ref[...], preferred_element_type=jnp.float32)
              + fcb_ref[...])
    m = jnp.max(logits, axis=1, keepdims=True)
    z = logits - m
    lse = jnp.log(jnp.sum(jnp.exp(z), axis=1, keepdims=True))
    out_ref[...] = z - lse


def _final(part, root, bias, fc_w, fc_b):
    n = root.shape[0]
    c = fc_w.shape[1]
    return pl.pallas_call(
        _final_body,
        out_shape=jax.ShapeDtypeStruct((n, c), jnp.float32),
    )(part, root, bias.reshape(1, H), fc_w, fc_b.reshape(1, c))


EDGE_BATCH = 80
SB = 8
BPT = 128


def _sc_aggregate_body(rows_per_tile, n_valid_batches,
                       xg_hbm, src_hbm, dst_hbm, w_hbm, out_hbm,
                       src_sb, dst_sb, dst_v, w_sb, g_v, msg_v, acc_sh, sem):
    cid = lax.axis_index("c")
    sid = lax.axis_index("s")
    wid = cid * NS + sid
    b = EDGE_BATCH

    zeros = jnp.zeros((LANES,), jnp.float32)
    ones = jnp.ones((LANES,), jnp.float32)

    def zero_row(r, carry):
        for c in range(AW // LANES):
            msg_v[r, pl.ds(c * LANES, LANES)] = zeros
        return carry

    lax.fori_loop(0, b, zero_row, 0)

    row0 = sid * rows_per_tile

    nfull, tail = divmod(rows_per_tile, EDGE_BATCH)
    for c2 in range(nfull):
        pltpu.sync_copy(msg_v,
                        acc_sh.at[pl.ds(row0 + c2 * EDGE_BATCH, EDGE_BATCH)])
    if tail:
        pltpu.sync_copy(msg_v.at[pl.ds(0, tail)],
                        acc_sh.at[pl.ds(row0 + nfull * EDGE_BATCH, tail)])

    def init_cnt_row(r, carry):
        msg_v[r, pl.ds(H, LANES)] = ones
        return carry

    lax.fori_loop(0, b, init_cnt_row, 0)
    plsc.subcore_barrier()

    def super_batch(i, carry):
        r0 = wid * BPT + i * SB
        pltpu.sync_copy(src_hbm.at[pl.ds(r0, SB)], src_sb)
        pltpu.sync_copy(dst_hbm.at[pl.ds(r0, SB)], dst_sb)
        pltpu.sync_copy(w_hbm.at[pl.ds(r0 * b * 4, SB * b * 4)], w_sb)
        for jj in range(SB):
            @pl.when(r0 + jj < n_valid_batches)
            def _batch():
                for t in range(b // LANES):
                    dst_v[pl.ds(t * LANES, LANES)] = (
                        dst_sb[jj, pl.ds(t * LANES, LANES)])
                pltpu.async_copy(xg_hbm.at[src_sb.at[jj]], g_v, sem).wait()

                def group(q, c2):
                    wv = w_sb[pl.ds((jj * b + q * 4) * 4, LANES)]
                    for t in range(4):
                        j = q * 4 + t
                        w0 = jnp.broadcast_to(wv[4 * t + 0], (LANES,))
                        w1 = jnp.broadcast_to(wv[4 * t + 1], (LANES,))
                        w2 = jnp.broadcast_to(wv[4 * t + 2], (LANES,))
                        for c in range(H // LANES):
                            g0 = g_v[j, pl.ds(c * LANES, LANES)]
                            g1 = g_v[j, pl.ds(H + c * LANES, LANES)]
                            g2 = g_v[j, pl.ds(2 * H + c * LANES, LANES)]
                            msg_v[j, pl.ds(c * LANES, LANES)] = (
                                g0 * w0 + g1 * w1 + g2 * w2)
                    return c2

                lax.fori_loop(0, b // 4, group, 0)
                pltpu.sync_copy(msg_v, acc_sh.at[dst_v], add=True)
        return carry

    lax.fori_loop(0, BPT // SB, super_batch, 0)
    plsc.subcore_barrier()

    pltpu.sync_copy(acc_sh.at[pl.ds(row0, rows_per_tile)],
                    out_hbm.at[cid, pl.ds(row0, rows_per_tile)])


def _sc_aggregate(xg, src, dst, w):
    n = xg.shape[0]
    e = src.shape[0]
    b = EDGE_BATCH
    n_valid_batches = e // b
    e_pad = NW * BPT * b
    src2d = jnp.pad(src, (0, e_pad - e)).reshape(e_pad // b, b)
    dst2d = jnp.pad(dst, (0, e_pad - e)).reshape(e_pad // b, b)
    w_pad = jnp.pad(w, ((0, e_pad - e), (0, 0))).reshape(-1)
    n_pad = -(-n // (NS * 8)) * (NS * 8)
    rows_per_tile = n_pad // NS
    mesh = plsc.VectorSubcoreMesh(core_axis_name="c", subcore_axis_name="s",
                                  num_cores=NC, num_subcores=NS)
    body = functools.partial(_sc_aggregate_body, rows_per_tile,
                             n_valid_batches)
    return pl.kernel(
        body,
        out_type=jax.ShapeDtypeStruct((NC, n_pad, AW), jnp.float32),
        mesh=mesh,
        scratch_types=[
            pltpu.VMEM((SB, b), jnp.int32),
            pltpu.VMEM((SB, b), jnp.int32),
            pltpu.VMEM((b,), jnp.int32),
            pltpu.VMEM((SB * b * (K + 1),), jnp.float32),
            pltpu.VMEM((b, XGW), jnp.float32),
            pltpu.VMEM((b, AW), jnp.float32),
            pltpu.VMEM_SHARED((n_pad, AW), jnp.float32),
            pltpu.SemaphoreType.DMA,
        ],
    )(xg, src2d, dst2d, w_pad)


def kernel(data, edge_index, edge_attr, g_W0, mu0, sigma0, root_W0, bias0,
           g_W1, mu1, sigma1, root_W1, bias1, fc_W, fc_b):
    src = edge_index[0]
    dst = edge_index[1]
    w0x, w1x = _edge_weights(edge_attr, mu0, sigma0, mu1, sigma1)
    xg0, root0 = _node_pre(data, g_W0, root_W0)
    part0 = _sc_aggregate(xg0, src, dst, w0x)
    xg1, root1 = _combine_pre(part0, root0, bias0, g_W1, root_W1)
    part1 = _sc_aggregate(xg1, src, dst, w1x)
    return _final(part1, root1, bias1, fc_W, fc_b)


# speedup vs baseline: 2.9850x; 1.5795x over previous
import functools

import jax
import jax.numpy as jnp
from jax import lax
from jax.experimental import pallas as pl
from jax.experimental.pallas import tpu as pltpu
from jax.experimental.pallas import tpu_sc as plsc

NC = 2
NS = 16
LANES = 16
NW = NC * NS

K = 3
H = 64
AW = 128
XGW = 256


def _edge_weights_body(a0_ref, a1_ref, mu0_ref, sig0_ref, mu1_ref, sig1_ref,
                       w0_ref, w1_ref):
    a0 = a0_ref[...]
    a1 = a1_ref[...]
    for mu_ref, sig_ref, out_ref in ((mu0_ref, sig0_ref, w0_ref),
                                     (mu1_ref, sig1_ref, w1_ref)):
        for k in range(K):
            d0 = a0 - mu_ref[k, 0]
            d1 = a1 - mu_ref[k, 1]
            q = -0.5 * (d0 * d0 / (1e-15 + sig_ref[k, 0] ** 2)
                        + d1 * d1 / (1e-15 + sig_ref[k, 1] ** 2))
            out_ref[:, k * EDGE_BATCH:(k + 1) * EDGE_BATCH] = jnp.exp(q)
        out_ref[:, K * EDGE_BATCH:] = jnp.ones(
            (a0.shape[0], EDGE_BATCH), jnp.float32)


def _edge_weights(edge_attr, mu0, sigma0, mu1, sigma1):
    e = edge_attr.shape[0]
    b = EDGE_BATCH
    rb = 200
    grid = e // (rb * b)
    a0 = edge_attr[:, 0].reshape(e // b, b)
    a1 = edge_attr[:, 1].reshape(e // b, b)
    smem = pl.BlockSpec(memory_space=pltpu.MemorySpace.SMEM)
    return pl.pallas_call(
        _edge_weights_body,
        grid=(grid,),
        in_specs=[
            pl.BlockSpec((rb, b), lambda i: (i, 0)),
            pl.BlockSpec((rb, b), lambda i: (i, 0)),
            smem, smem, smem, smem,
        ],
        out_specs=[
            pl.BlockSpec((rb, 4 * b), lambda i: (i, 0)),
            pl.BlockSpec((rb, 4 * b), lambda i: (i, 0)),
        ],
        out_shape=[
            jax.ShapeDtypeStruct((e // b, 4 * b), jnp.float32),
            jax.ShapeDtypeStruct((e // b, 4 * b), jnp.float32),
        ],
    )(a0, a1, mu0, sigma0, mu1, sigma1)


def _node_pre_body(x_ref, gw_ref, rw_ref, xg_ref, root_ref):
    x = x_ref[...]
    n = x.shape[0]
    xg_ref[:, 0:K * H] = jnp.dot(x, gw_ref[...],
                                 preferred_element_type=jnp.float32)
    xg_ref[:, K * H:] = jnp.zeros((n, XGW - K * H), jnp.float32)
    root_ref[...] = jnp.dot(x, rw_ref[...], preferred_element_type=jnp.float32)


def _node_pre(x, g_w, root_w):
    n = x.shape[0]
    return pl.pallas_call(
        _node_pre_body,
        out_shape=[
            jax.ShapeDtypeStruct((n, XGW), jnp.float32),
            jax.ShapeDtypeStruct((n, H), jnp.float32),
        ],
    )(x, g_w, root_w)


def _combine_pre_body(part_ref, root_ref, bias_ref, gw_ref, rw_ref,
                      xg_ref, root1_ref):
    n = root_ref.shape[0]
    s = part_ref[0, :n] + part_ref[1, :n]
    cnt = jnp.maximum(s[:, H:H + 1], 1.0)
    aggr = s[:, 0:H] / cnt
    x1 = jnp.maximum(aggr + root_ref[...] + bias_ref[...], 0.0)
    xg_ref[:, 0:K * H] = jnp.dot(x1, gw_ref[...],
                                 preferred_element_type=jnp.float32)
    xg_ref[:, K * H:] = jnp.zeros((n, XGW - K * H), jnp.float32)
    root1_ref[...] = jnp.dot(x1, rw_ref[...],
                             preferred_element_type=jnp.float32)


def _combine_pre(part, root, bias, g_w, root_w):
    n = root.shape[0]
    return pl.pallas_call(
        _combine_pre_body,
        out_shape=[
            jax.ShapeDtypeStruct((n, XGW), jnp.float32),
            jax.ShapeDtypeStruct((n, H), jnp.float32),
        ],
    )(part, root, bias.reshape(1, H), g_w, root_w)


def _final_body(part_ref, root_ref, bias_ref, fcw_ref, fcb_ref, out_ref):
    n = root_ref.shape[0]
    s = part_ref[0, :n] + part_ref[1, :n]
    cnt = jnp.maximum(s[:, H:H + 1], 1.0)
    aggr = s[:, 0:H] / cnt
    x2 = jnp.maximum(aggr + root_ref[...] + bias_ref[...], 0.0)
    logits = (jnp.dot(x2, fcw_ref[...], preferred_element_type=jnp.float32)
              + fcb_ref[...])
    m = jnp.max(logits, axis=1, keepdims=True)
    z = logits - m
    lse = jnp.log(jnp.sum(jnp.exp(z), axis=1, keepdims=True))
    out_ref[...] = z - lse


def _final(part, root, bias, fc_w, fc_b):
    n = root.shape[0]
    c = fc_w.shape[1]
    return pl.pallas_call(
        _final_body,
        out_shape=jax.ShapeDtypeStruct((n, c), jnp.float32),
    )(part, root, bias.reshape(1, H), fc_w, fc_b.reshape(1, c))


EDGE_BATCH = 80
SB = 8
BPT = 128


def _sc_aggregate_body(rows_per_tile, n_valid_batches,
                       xg_hbm, src_hbm, dst_hbm, w_hbm, out_hbm,
                       src_sb, dst_sb, dst_v, w_sb, g_v, msg_v, acc_sh, sem):
    cid = lax.axis_index("c")
    sid = lax.axis_index("s")
    wid = cid * NS + sid
    b = EDGE_BATCH

    zeros = jnp.zeros((LANES,), jnp.float32)
    ones = jnp.ones((LANES,), jnp.float32)

    def zero_row(r, carry):
        for c in range(AW // LANES):
            msg_v[r, pl.ds(c * LANES, LANES)] = zeros
        return carry

    lax.fori_loop(0, b, zero_row, 0)

    row0 = sid * rows_per_tile

    nfull, tail = divmod(rows_per_tile, EDGE_BATCH)
    for c2 in range(nfull):
        pltpu.sync_copy(msg_v,
                        acc_sh.at[pl.ds(row0 + c2 * EDGE_BATCH, EDGE_BATCH)])
    if tail:
        pltpu.sync_copy(msg_v.at[pl.ds(0, tail)],
                        acc_sh.at[pl.ds(row0 + nfull * EDGE_BATCH, tail)])

    def init_cnt_row(r, carry):
        msg_v[r, pl.ds(H, LANES)] = ones
        return carry

    lax.fori_loop(0, b, init_cnt_row, 0)
    plsc.subcore_barrier()

    def super_batch(i, carry):
        r0 = wid * BPT + i * SB
        pltpu.sync_copy(src_hbm.at[pl.ds(r0, SB)], src_sb)
        pltpu.sync_copy(dst_hbm.at[pl.ds(r0, SB)], dst_sb)
        pltpu.sync_copy(w_hbm.at[pl.ds(r0, SB)], w_sb)
        for jj in range(SB):
            @pl.when(r0 + jj < n_valid_batches)
            def _batch():
                for t in range(b // LANES):
                    dst_v[pl.ds(t * LANES, LANES)] = (
                        dst_sb[jj, pl.ds(t * LANES, LANES)])
                pltpu.async_copy(xg_hbm.at[src_sb.at[jj]], g_v, sem).wait()

                def group(q, c2):
                    v0 = w_sb[jj, pl.ds(q * LANES, LANES)]
                    v1 = w_sb[jj, pl.ds(b + q * LANES, LANES)]
                    v2 = w_sb[jj, pl.ds(2 * b + q * LANES, LANES)]
                    for t in range(LANES):
                        j = q * LANES + t
                        w0 = jnp.broadcast_to(v0[t], (LANES,))
                        w1 = jnp.broadcast_to(v1[t], (LANES,))
                        w2 = jnp.broadcast_to(v2[t], (LANES,))
                        for c in range(H // LANES):
                            g0 = g_v[j, pl.ds(c * LANES, LANES)]
                            g1 = g_v[j, pl.ds(H + c * LANES, LANES)]
                            g2 = g_v[j, pl.ds(2 * H + c * LANES, LANES)]
                            msg_v[j, pl.ds(c * LANES, LANES)] = (
                                g0 * w0 + g1 * w1 + g2 * w2)
                    return c2

                lax.fori_loop(0, b // LANES, group, 0)
                pltpu.sync_copy(msg_v, acc_sh.at[dst_v], add=True)
        return carry

    lax.fori_loop(0, BPT // SB, super_batch, 0)
    plsc.subcore_barrier()

    pltpu.sync_copy(acc_sh.at[pl.ds(row0, rows_per_tile)],
                    out_hbm.at[cid, pl.ds(row0, rows_per_tile)])


def _sc_aggregate(xg, src, dst, w):
    n = xg.shape[0]
    e = src.shape[0]
    b = EDGE_BATCH
    n_valid_batches = e // b
    e_pad = NW * BPT * b
    src2d = jnp.pad(src, (0, e_pad - e)).reshape(e_pad // b, b)
    dst2d = jnp.pad(dst, (0, e_pad - e)).reshape(e_pad // b, b)
    w_pad = jnp.pad(w, ((0, e_pad // b - w.shape[0]), (0, 0)))
    n_pad = -(-n // (NS * 8)) * (NS * 8)
    rows_per_tile = n_pad // NS
    mesh = plsc.VectorSubcoreMesh(core_axis_name="c", subcore_axis_name="s",
                                  num_cores=NC, num_subcores=NS)
    body = functools.partial(_sc_aggregate_body, rows_per_tile,
                             n_valid_batches)
    return pl.kernel(
        body,
        out_type=jax.ShapeDtypeStruct((NC, n_pad, AW), jnp.float32),
        mesh=mesh,
        scratch_types=[
            pltpu.VMEM((SB, b), jnp.int32),
            pltpu.VMEM((SB, b), jnp.int32),
            pltpu.VMEM((b,), jnp.int32),
            pltpu.VMEM((SB, b * 4), jnp.float32),
            pltpu.VMEM((b, XGW), jnp.float32),
            pltpu.VMEM((b, AW), jnp.float32),
            pltpu.VMEM_SHARED((n_pad, AW), jnp.float32),
            pltpu.SemaphoreType.DMA,
        ],
    )(xg, src2d, dst2d, w_pad)


def kernel(data, edge_index, edge_attr, g_W0, mu0, sigma0, root_W0, bias0,
           g_W1, mu1, sigma1, root_W1, bias1, fc_W, fc_b):
    src = edge_index[0]
    dst = edge_index[1]
    w0x, w1x = _edge_weights(edge_attr, mu0, sigma0, mu1, sigma1)
    xg0, root0 = _node_pre(data, g_W0, root_W0)
    part0 = _sc_aggregate(xg0, src, dst, w0x)
    xg1, root1 = _combine_pre(part0, root0, bias0, g_W1, root_W1)
    part1 = _sc_aggregate(xg1, src, dst, w1x)
    return _final(part1, root1, bias1, fc_W, fc_b)


# speedup vs baseline: 3.2164x; 1.0775x over previous
import functools

import jax
import jax.numpy as jnp
from jax import lax
from jax.experimental import pallas as pl
from jax.experimental.pallas import tpu as pltpu
from jax.experimental.pallas import tpu_sc as plsc

NC = 2
NS = 16
LANES = 16
NW = NC * NS

K = 3
H = 64
AW = 128
XGW = 256


def _edge_weights_body(a0_ref, a1_ref, mu0_ref, sig0_ref, mu1_ref, sig1_ref,
                       w0_ref, w1_ref):
    a0 = a0_ref[...]
    a1 = a1_ref[...]
    for mu_ref, sig_ref, out_ref in ((mu0_ref, sig0_ref, w0_ref),
                                     (mu1_ref, sig1_ref, w1_ref)):
        for k in range(K):
            d0 = a0 - mu_ref[k, 0]
            d1 = a1 - mu_ref[k, 1]
            q = -0.5 * (d0 * d0 / (1e-15 + sig_ref[k, 0] ** 2)
                        + d1 * d1 / (1e-15 + sig_ref[k, 1] ** 2))
            out_ref[:, k * EDGE_BATCH:(k + 1) * EDGE_BATCH] = jnp.exp(q)
        out_ref[:, K * EDGE_BATCH:] = jnp.ones(
            (a0.shape[0], EDGE_BATCH), jnp.float32)


def _edge_weights(edge_attr, mu0, sigma0, mu1, sigma1):
    e = edge_attr.shape[0]
    b = EDGE_BATCH
    rb = 200
    grid = e // (rb * b)
    a0 = edge_attr[:, 0].reshape(e // b, b)
    a1 = edge_attr[:, 1].reshape(e // b, b)
    smem = pl.BlockSpec(memory_space=pltpu.MemorySpace.SMEM)
    return pl.pallas_call(
        _edge_weights_body,
        grid=(grid,),
        in_specs=[
            pl.BlockSpec((rb, b), lambda i: (i, 0)),
            pl.BlockSpec((rb, b), lambda i: (i, 0)),
            smem, smem, smem, smem,
        ],
        out_specs=[
            pl.BlockSpec((rb, 4 * b), lambda i: (i, 0)),
            pl.BlockSpec((rb, 4 * b), lambda i: (i, 0)),
        ],
        out_shape=[
            jax.ShapeDtypeStruct((e // b, 4 * b), jnp.float32),
            jax.ShapeDtypeStruct((e // b, 4 * b), jnp.float32),
        ],
    )(a0, a1, mu0, sigma0, mu1, sigma1)


def _node_pre_body(x_ref, gw_ref, rw_ref, xg_ref, root_ref):
    x = x_ref[...]
    n = x.shape[0]
    xg_ref[:, 0:K * H] = jnp.dot(x, gw_ref[...],
                                 preferred_element_type=jnp.float32)
    xg_ref[:, K * H:] = jnp.zeros((n, XGW - K * H), jnp.float32)
    root_ref[...] = jnp.dot(x, rw_ref[...], preferred_element_type=jnp.float32)


def _node_pre(x, g_w, root_w):
    n = x.shape[0]
    return pl.pallas_call(
        _node_pre_body,
        out_shape=[
            jax.ShapeDtypeStruct((n, XGW), jnp.float32),
            jax.ShapeDtypeStruct((n, H), jnp.float32),
        ],
    )(x, g_w, root_w)


def _combine_pre_body(part_ref, root_ref, bias_ref, gw_ref, rw_ref,
                      xg_ref, root1_ref):
    n = root_ref.shape[0]
    s = part_ref[0, :n] + part_ref[1, :n]
    cnt = jnp.maximum(s[:, H:H + 1], 1.0)
    aggr = s[:, 0:H] / cnt
    x1 = jnp.maximum(aggr + root_ref[...] + bias_ref[...], 0.0)
    xg_ref[:, 0:K * H] = jnp.dot(x1, gw_ref[...],
                                 preferred_element_type=jnp.float32)
    xg_ref[:, K * H:] = jnp.zeros((n, XGW - K * H), jnp.float32)
    root1_ref[...] = jnp.dot(x1, rw_ref[...],
                             preferred_element_type=jnp.float32)


def _combine_pre(part, root, bias, g_w, root_w):
    n = root.shape[0]
    return pl.pallas_call(
        _combine_pre_body,
        out_shape=[
            jax.ShapeDtypeStruct((n, XGW), jnp.float32),
            jax.ShapeDtypeStruct((n, H), jnp.float32),
        ],
    )(part, root, bias.reshape(1, H), g_w, root_w)


def _final_body(part_ref, root_ref, bias_ref, fcw_ref, fcb_ref, out_ref):
    n = root_ref.shape[0]
    s = part_ref[0, :n] + part_ref[1, :n]
    cnt = jnp.maximum(s[:, H:H + 1], 1.0)
    aggr = s[:, 0:H] / cnt
    x2 = jnp.maximum(aggr + root_ref[...] + bias_ref[...], 0.0)
    logits = (jnp.dot(x2, fcw_ref[...], preferred_element_type=jnp.float32)
              + fcb_ref[...])
    m = jnp.max(logits, axis=1, keepdims=True)
    z = logits - m
    lse = jnp.log(jnp.sum(jnp.exp(z), axis=1, keepdims=True))
    out_ref[...] = z - lse


def _final(part, root, bias, fc_w, fc_b):
    n = root.shape[0]
    c = fc_w.shape[1]
    return pl.pallas_call(
        _final_body,
        out_shape=jax.ShapeDtypeStruct((n, c), jnp.float32),
    )(part, root, bias.reshape(1, H), fc_w, fc_b.reshape(1, c))


EDGE_BATCH = 80
SB = 8
BPT = 128


def _sc_aggregate_body(rows_per_tile, n_valid_batches,
                       xg_hbm, src_hbm, dst_hbm, w_hbm, out_hbm,
                       src_sb, dst_sb, dst_v, w_sb, g_v, msg_v, acc_sh,
                       sem, sem2):
    cid = lax.axis_index("c")
    sid = lax.axis_index("s")
    wid = cid * NS + sid
    b = EDGE_BATCH

    zeros = jnp.zeros((LANES,), jnp.float32)
    ones = jnp.ones((LANES,), jnp.float32)

    def zero_row(r, carry):
        for c in range(AW // LANES):
            msg_v[r, pl.ds(c * LANES, LANES)] = zeros
        return carry

    lax.fori_loop(0, b, zero_row, 0)

    row0 = sid * rows_per_tile

    nfull, tail = divmod(rows_per_tile, EDGE_BATCH)
    for c2 in range(nfull):
        pltpu.sync_copy(msg_v,
                        acc_sh.at[pl.ds(row0 + c2 * EDGE_BATCH, EDGE_BATCH)])
    if tail:
        pltpu.sync_copy(msg_v.at[pl.ds(0, tail)],
                        acc_sh.at[pl.ds(row0 + nfull * EDGE_BATCH, tail)])

    def init_cnt_row(r, carry):
        msg_v[r, pl.ds(H, LANES)] = ones
        return carry

    lax.fori_loop(0, b, init_cnt_row, 0)
    plsc.subcore_barrier()

    def super_batch(i, carry):
        r0 = wid * BPT + i * SB

        @pl.when(r0 < n_valid_batches)
        def _sb():
            pltpu.sync_copy(src_hbm.at[pl.ds(r0, SB)], src_sb)
            pltpu.sync_copy(dst_hbm.at[pl.ds(r0, SB)], dst_sb)
            pltpu.sync_copy(w_hbm.at[pl.ds(r0, SB)], w_sb)


            def fire(jj2):
                return pltpu.async_copy(xg_hbm.at[src_sb.at[jj2]], g_v, sem)

            cps = fire(0)
            for jj in range(SB):
                for t in range(b // LANES):
                    dst_v[pl.ds(t * LANES, LANES)] = (
                        dst_sb[jj, pl.ds(t * LANES, LANES)])

                def group(q, c2):
                    v0 = w_sb[jj, pl.ds(q * LANES, LANES)]
                    v1 = w_sb[jj, pl.ds(b + q * LANES, LANES)]
                    v2 = w_sb[jj, pl.ds(2 * b + q * LANES, LANES)]
                    for t in range(LANES):
                        j = q * LANES + t
                        w0 = jnp.broadcast_to(v0[t], (LANES,))
                        w1 = jnp.broadcast_to(v1[t], (LANES,))
                        w2 = jnp.broadcast_to(v2[t], (LANES,))
                        for c in range(H // LANES):
                            g0 = g_v[j, pl.ds(c * LANES, LANES)]
                            g1 = g_v[j, pl.ds(H + c * LANES, LANES)]
                            g2 = g_v[j, pl.ds(2 * H + c * LANES, LANES)]
                            msg_v[j, pl.ds(c * LANES, LANES)] = (
                                g0 * w0 + g1 * w1 + g2 * w2)
                    return c2

                cps.wait()
                lax.fori_loop(0, b // LANES, group, 0)
                if jj + 1 < SB:
                    cps = fire(jj + 1)
                pltpu.sync_copy(msg_v, acc_sh.at[dst_v], add=True)
        return carry

    lax.fori_loop(0, BPT // SB, super_batch, 0)
    plsc.subcore_barrier()

    pltpu.sync_copy(acc_sh.at[pl.ds(row0, rows_per_tile)],
                    out_hbm.at[cid, pl.ds(row0, rows_per_tile)])


def _sc_aggregate(xg, src, dst, w):
    n = xg.shape[0]
    e = src.shape[0]
    b = EDGE_BATCH
    n_valid_batches = e // b
    e_pad = NW * BPT * b
    src2d = jnp.pad(src, (0, e_pad - e)).reshape(e_pad // b, b)
    dst2d = jnp.pad(dst, (0, e_pad - e)).reshape(e_pad // b, b)
    w_pad = jnp.pad(w, ((0, e_pad // b - w.shape[0]), (0, 0)))
    n_pad = -(-n // (NS * 8)) * (NS * 8)
    rows_per_tile = n_pad // NS
    mesh = plsc.VectorSubcoreMesh(core_axis_name="c", subcore_axis_name="s",
                                  num_cores=NC, num_subcores=NS)
    body = functools.partial(_sc_aggregate_body, rows_per_tile,
                             n_valid_batches)
    return pl.kernel(
        body,
        out_type=jax.ShapeDtypeStruct((NC, n_pad, AW), jnp.float32),
        mesh=mesh,
        scratch_types=[
            pltpu.VMEM((SB, b), jnp.int32),
            pltpu.VMEM((SB, b), jnp.int32),
            pltpu.VMEM((b,), jnp.int32),
            pltpu.VMEM((SB, b * 4), jnp.float32),
            pltpu.VMEM((b, XGW), jnp.float32),
            pltpu.VMEM((b, AW), jnp.float32),
            pltpu.VMEM_SHARED((n_pad, AW), jnp.float32),
            pltpu.SemaphoreType.DMA,
            pltpu.SemaphoreType.DMA,
        ],
    )(xg, src2d, dst2d, w_pad)


def kernel(data, edge_index, edge_attr, g_W0, mu0, sigma0, root_W0, bias0,
           g_W1, mu1, sigma1, root_W1, bias1, fc_W, fc_b):
    src = edge_index[0]
    dst = edge_index[1]
    w0x, w1x = _edge_weights(edge_attr, mu0, sigma0, mu1, sigma1)
    xg0, root0 = _node_pre(data, g_W0, root_W0)
    part0 = _sc_aggregate(xg0, src, dst, w0x)
    xg1, root1 = _combine_pre(part0, root0, bias0, g_W1, root_W1)
    part1 = _sc_aggregate(xg1, src, dst, w1x)
    return _final(part1, root1, bias1, fc_W, fc_b)
